# baseline (speedup 1.0000x reference)
import jax
import jax.numpy as jnp
from jax.experimental import pallas as pl
from jax.experimental.pallas import tpu as pltpu

_SIGMA = 0.1
_NCORES = 2
_CH = 512


def _round_up(v, m):
    return ((v + m - 1) // m) * m


def _make_kernel(bt, nc):
    def _kern(x_hbm, w_hbm, nw_hbm, b_ref, nb_ref, o_hbm,
              w_vm, nw_vm, weff_ref, beff_ref, x_buf, o_buf,
              w_sem, in_sem, out_sem):
        base = pl.program_id(0) * bt

        def in_copy(slot, c):
            return pltpu.make_async_copy(
                x_hbm.at[pl.ds(base + c * _CH, _CH), :],
                x_buf.at[slot], in_sem.at[slot])

        def out_copy(slot, c):
            return pltpu.make_async_copy(
                o_buf.at[slot],
                o_hbm.at[pl.ds(base + c * _CH, _CH), :], out_sem.at[slot])

        w_copy = pltpu.make_async_copy(w_hbm, w_vm, w_sem.at[0])
        nw_copy = pltpu.make_async_copy(nw_hbm, nw_vm, w_sem.at[1])
        w_copy.start()
        nw_copy.start()
        in_copy(0, 0).start()
        if nc > 1:
            in_copy(1, 1).start()

        w_copy.wait()
        nw_copy.wait()
        weff_ref[...] = (w_vm[...] + _SIGMA * nw_vm[...]).astype(jnp.bfloat16)
        beff_ref[...] = b_ref[...] + _SIGMA * nb_ref[...]

        for c in range(nc):
            cur = c % 2
            in_copy(cur, c).wait()
            if c >= 2:
                out_copy(cur, c - 2).wait()
            o_buf[cur] = (
                jnp.dot(x_buf[cur].astype(jnp.bfloat16), weff_ref[...],
                        preferred_element_type=jnp.float32)
                + beff_ref[...]
            )
            out_copy(cur, c).start()
            if c + 2 < nc:
                in_copy(cur, c + 2).start()

        if nc >= 2:
            out_copy((nc - 2) % 2, nc - 2).wait()
        out_copy((nc - 1) % 2, nc - 1).wait()

    return _kern


def kernel(x, w_t, bias2d, noise_w_t, noise_b2d):
    B, K = x.shape
    Kw, N = w_t.shape
    assert K == Kw

    bt = _round_up(B, _CH * _NCORES) // _NCORES
    Bp = bt * _NCORES
    x_p = x if Bp == B else jnp.pad(x, ((0, Bp - B), (0, 0)))
    nc = bt // _CH

    anyspace = pl.BlockSpec(memory_space=pltpu.MemorySpace.HBM)
    vmem = pl.BlockSpec(memory_space=pltpu.MemorySpace.VMEM)

    out = pl.pallas_call(
        _make_kernel(bt, nc),
        grid=(_NCORES,),
        in_specs=[anyspace, anyspace, anyspace, vmem, vmem],
        out_specs=anyspace,
        out_shape=jax.ShapeDtypeStruct((Bp, N), jnp.float32),
        scratch_shapes=[
            pltpu.VMEM((K, N), jnp.float32),
            pltpu.VMEM((K, N), jnp.float32),
            pltpu.VMEM((K, N), jnp.bfloat16),
            pltpu.VMEM((1, N), jnp.float32),
            pltpu.VMEM((2, _CH, K), jnp.float32),
            pltpu.VMEM((2, _CH, N), jnp.float32),
            pltpu.SemaphoreType.DMA((2,)),
            pltpu.SemaphoreType.DMA((2,)),
            pltpu.SemaphoreType.DMA((2,)),
        ],
        compiler_params=pltpu.CompilerParams(
            dimension_semantics=("parallel",),
            vmem_limit_bytes=48 << 20,
        ),
    )(x_p, w_t, noise_w_t, bias2d, noise_b2d)

    return out if Bp == B else out[:B]


# speedup vs baseline: 1.2375x; 1.2375x over previous
import jax
import jax.numpy as jnp
from jax.experimental import pallas as pl
from jax.experimental.pallas import tpu as pltpu

_SIGMA = 0.1
_NCORES = 2
_CH = 512


def _round_up(v, m):
    return ((v + m - 1) // m) * m


def _make_kernel(bt, nc):
    def _kern(x_ref, w_ref, nw_ref, b_ref, nb_ref, o_hbm,
              weff_ref, beff_ref, o_vm, out_sem):
        base = pl.program_id(0) * bt

        weff_ref[...] = (w_ref[...] + _SIGMA * nw_ref[...]).astype(jnp.bfloat16)
        beff_ref[...] = b_ref[...] + _SIGMA * nb_ref[...]

        for c in range(nc):
            sl = pl.ds(c * _CH, _CH)
            o_vm[sl, :] = (
                jnp.dot(x_ref[sl, :].astype(jnp.bfloat16), weff_ref[...],
                        preferred_element_type=jnp.float32)
                + beff_ref[...]
            )
            pltpu.make_async_copy(
                o_vm.at[sl, :],
                o_hbm.at[pl.ds(base + c * _CH, _CH), :],
                out_sem.at[c]).start()

        for c in range(nc):
            pltpu.make_async_copy(
                o_vm.at[pl.ds(c * _CH, _CH), :],
                o_hbm.at[pl.ds(base + c * _CH, _CH), :],
                out_sem.at[c]).wait()

    return _kern


def kernel(x, w_t, bias2d, noise_w_t, noise_b2d):
    B, K = x.shape
    Kw, N = w_t.shape
    assert K == Kw

    bt = _round_up(B, _CH * _NCORES) // _NCORES
    Bp = bt * _NCORES
    x_p = x if Bp == B else jnp.pad(x, ((0, Bp - B), (0, 0)))
    nc = bt // _CH

    out = pl.pallas_call(
        _make_kernel(bt, nc),
        grid=(_NCORES,),
        in_specs=[
            pl.BlockSpec((bt, K), lambda i: (i, 0)),
            pl.BlockSpec((K, N), lambda i: (0, 0)),
            pl.BlockSpec((K, N), lambda i: (0, 0)),
            pl.BlockSpec((1, N), lambda i: (0, 0)),
            pl.BlockSpec((1, N), lambda i: (0, 0)),
        ],
        out_specs=pl.BlockSpec(memory_space=pltpu.MemorySpace.HBM),
        out_shape=jax.ShapeDtypeStruct((Bp, N), jnp.float32),
        scratch_shapes=[
            pltpu.VMEM((K, N), jnp.bfloat16),
            pltpu.VMEM((1, N), jnp.float32),
            pltpu.VMEM((bt, N), jnp.float32),
            pltpu.SemaphoreType.DMA((nc,)),
        ],
        compiler_params=pltpu.CompilerParams(
            dimension_semantics=("parallel",),
            vmem_limit_bytes=48 << 20,
        ),
    )(x_p, w_t, noise_w_t, bias2d, noise_b2d)

    return out if Bp == B else out[:B]
